# baseline (speedup 1.0000x reference)
import functools

import numpy as np
import jax
import jax.numpy as jnp
from jax import lax
from jax.experimental import pallas as pl
from jax.experimental.pallas import tpu as pltpu
from jax.experimental.pallas import tpu_sc as plsc

_N_LEVELS = 16
_BASE_RES = 16
_MAX_PARAMS = 2 ** 19
_B_SCALE = 1.3195079565048218
_P1 = int(np.uint32(2654435761).view(np.int32))
_P2 = int(np.uint32(805459861).view(np.int32))
_HASH_MASK = _MAX_PARAMS - 1


def _levels():
    out = []
    off = 0
    for i in range(_N_LEVELS):
        scale = _BASE_RES * np.exp(i * np.log(_B_SCALE)) - 1.0
        res = int(np.ceil(scale)) + 1
        params = res ** 3
        if params % 8 != 0:
            params = ((params + 7) // 8) * 8
        params = min(_MAX_PARAMS, params)
        dense = res ** 3 <= params
        out.append((np.float32(scale), res, params, off, dense))
        off += params
    return out, off


_LEVEL_META, _TOTAL_ROWS = _levels()

_NW = 32
_B = 512
_G = _B // 16

_N_LOCAL = 3
_LOCAL_FLOATS = 2 * sum(m[2] for m in _LEVEL_META[:_N_LOCAL])
_LOCAL_PAD = (-(-_LOCAL_FLOATS // 8)) * 8


def _encode_body(px_hbm, py_hbm, pz_hbm, tab_hbm, tab1_hbm, out_hbm,
                 xs_v, ys_v, zs_v,
                 idx_a, idx_b, w_a, w_b, w_c,
                 rows0_a, rows1_a, rows0_b, rows1_b,
                 out_v, tab01_v, sem_a, sem_b):
    wid = lax.axis_index("s") * 2 + lax.axis_index("c")
    pltpu.sync_copy(tab_hbm.at[pl.ds(0, _LOCAL_PAD)], tab01_v)

    n = px_hbm.shape[0]
    npw = n // _NW
    nchunks = npw // _B
    lane = lax.iota(jnp.int32, 16)
    idx_bufs = (idx_a, idx_b)
    w_bufs = (w_a, w_b, w_c)
    row_bufs = ((rows0_a, rows1_a), (rows0_b, rows1_b))
    sems = (sem_a, sem_b)

    def chunk_body(i, _):
        base = wid * npw + i * _B
        pltpu.sync_copy(px_hbm.at[pl.ds(base, _B)], xs_v)
        pltpu.sync_copy(py_hbm.at[pl.ds(base, _B)], ys_v)
        pltpu.sync_copy(pz_hbm.at[pl.ds(base, _B)], zs_v)

        def corner_setup(level, g):
            scale, res, size, off, dense = _LEVEL_META[level]
            s = g * 16
            x = xs_v[pl.ds(s, 16)]
            y = ys_v[pl.ds(s, 16)]
            z = zs_v[pl.ds(s, 16)]
            fscale = jnp.float32(scale)
            pxv = x * fscale + jnp.float32(0.5)
            pyv = y * fscale + jnp.float32(0.5)
            pzv = z * fscale + jnp.float32(0.5)
            ix = pxv.astype(jnp.int32)
            iy = pyv.astype(jnp.int32)
            iz = pzv.astype(jnp.int32)
            fx = pxv - ix.astype(jnp.float32)
            fy = pyv - iy.astype(jnp.float32)
            fz = pzv - iz.astype(jnp.float32)
            wx = (jnp.float32(1.0) - fx, fx)
            wy = (jnp.float32(1.0) - fy, fy)
            wz = (jnp.float32(1.0) - fz, fz)
            if dense:
                res2 = res * res
                cs = ((ix + off, ix + off + 1),
                      (iy * res, iy * res + res),
                      (iz * res2, iz * res2 + res2))
                lim = size + off
            else:
                cs = ((ix, ix + 1),
                      (iy * _P1, iy * _P1 + _P1),
                      (iz * _P2, iz * _P2 + _P2))
                lim = None
            return cs, lim, (wx, wy, wz), s, off, size, dense

        def corner_row(cs, lim, off, size, dense, corner):
            bx, by, bz = corner & 1, (corner >> 1) & 1, (corner >> 2) & 1
            if dense:
                h = cs[0][bx] + cs[1][by] + cs[2][bz]
                row = jnp.where(h >= lim, h - size, h)
            else:
                h = cs[0][bx] ^ cs[1][by] ^ cs[2][bz]
                row = (h & _HASH_MASK) + off
            return row

        def corner_w(ws, corner):
            bx, by, bz = corner & 1, (corner >> 1) & 1, (corner >> 2) & 1
            return ws[0][bx] * ws[1][by] * ws[2][bz]

        def idx_part(level, g):
            idx_v = idx_bufs[level % 2]
            w_v = w_bufs[level % 3]
            cs, lim, ws, s, off, size, dense = corner_setup(level, g)
            for corner in range(8):
                row = corner_row(cs, lim, off, size, dense, corner)
                idx_v[pl.ds(corner * _B + s, 16)] = row * 2
                w_v[pl.ds(corner * _B + s, 16)] = corner_w(ws, corner)

        def red_part(level, g):
            rows0_v, rows1_v = row_bufs[level % 2]
            w_v = w_bufs[level % 3]
            s = g * 16
            acc0 = jnp.zeros((16,), jnp.float32)
            acc1 = jnp.zeros((16,), jnp.float32)
            for corner in range(8):
                f0 = rows0_v[pl.ds(corner * _B + s, 16)]
                f1 = rows1_v[pl.ds(corner * _B + s, 16)]
                w = w_v[pl.ds(corner * _B + s, 16)]
                acc0 = acc0 + w * f0
                acc1 = acc1 + w * f1
            oidx = (s + lane) * 32 + (2 * level)
            plsc.store_scatter(out_v, [oidx], acc0)
            plsc.store_scatter(out_v, [oidx + 1], acc1)

        def local_part(level, g):
            cs, lim, ws, s, off, size, dense = corner_setup(level, g)
            acc0 = jnp.zeros((16,), jnp.float32)
            acc1 = jnp.zeros((16,), jnp.float32)
            for corner in range(8):
                row = corner_row(cs, lim, off, size, dense, corner)
                el0 = row * 2
                w = corner_w(ws, corner)
                f0 = plsc.load_gather(tab01_v, [el0])
                f1 = plsc.load_gather(tab01_v, [el0 + 1])
                acc0 = acc0 + w * f0
                acc1 = acc1 + w * f1
            oidx = (s + lane) * 32 + (2 * level)
            plsc.store_scatter(out_v, [oidx], acc0)
            plsc.store_scatter(out_v, [oidx + 1], acc1)

        def issue(level):
            b = level % 2
            cp0 = pltpu.async_copy(tab_hbm.at[idx_bufs[b]], row_bufs[b][0],
                                   sems[b])
            cp1 = pltpu.async_copy(tab1_hbm.at[idx_bufs[b]], row_bufs[b][1],
                                   sems[b])
            return (cp0, cp1)

        def run_loop(red_level, idx_level, local_level=None):
            def _b(g):
                if red_level is not None:
                    red_part(red_level, g)
                if idx_level is not None:
                    idx_part(idx_level, g)
                if local_level is not None:
                    local_part(local_level, g)

            plsc.parallel_loop(0, _G, 1, unroll=2)(_b)

        run_loop(None, _N_LOCAL)
        cp = issue(_N_LOCAL)
        run_loop(None, _N_LOCAL + 1)
        cp_next = issue(_N_LOCAL + 1)
        for lv in range(_N_LOCAL):
            run_loop(None, None, lv)
        for level in range(_N_LOCAL, _N_LEVELS):
            cp[0].wait()
            cp[1].wait()
            run_loop(level, level + 2 if level + 2 < _N_LEVELS else None)
            if level + 2 < _N_LEVELS:
                cp = cp_next
                cp_next = issue(level + 2)
            elif level + 1 < _N_LEVELS:
                cp = cp_next

        pltpu.sync_copy(out_v, out_hbm.at[pl.ds(base * 32, _B * 32)])
        return 0

    lax.fori_loop(0, nchunks, chunk_body, 0, unroll=False)


def kernel(positions, hash_table):
    n = positions.shape[0]
    px = positions[:, 0]
    py = positions[:, 1]
    pz = positions[:, 2]
    tab1 = jnp.pad(hash_table[1:], (0, 1))

    mesh = plsc.VectorSubcoreMesh(core_axis_name="c", subcore_axis_name="s")
    run = functools.partial(
        pl.kernel,
        mesh=mesh,
        compiler_params=pltpu.CompilerParams(needs_layout_passes=False),
        out_type=jax.ShapeDtypeStruct((n * 32,), jnp.float32),
        scratch_types=[
            pltpu.VMEM((_B,), jnp.float32),
            pltpu.VMEM((_B,), jnp.float32),
            pltpu.VMEM((_B,), jnp.float32),
            pltpu.VMEM((8 * _B,), jnp.int32),
            pltpu.VMEM((8 * _B,), jnp.int32),
            pltpu.VMEM((8 * _B,), jnp.float32),
            pltpu.VMEM((8 * _B,), jnp.float32),
            pltpu.VMEM((8 * _B,), jnp.float32),
            pltpu.VMEM((8 * _B,), jnp.float32),
            pltpu.VMEM((8 * _B,), jnp.float32),
            pltpu.VMEM((8 * _B,), jnp.float32),
            pltpu.VMEM((8 * _B,), jnp.float32),
            pltpu.VMEM((_B * 32,), jnp.float32),
            pltpu.VMEM((_LOCAL_PAD,), jnp.float32),
            pltpu.SemaphoreType.DMA,
            pltpu.SemaphoreType.DMA,
        ],
    )(_encode_body)
    out = run(px, py, pz, hash_table, tab1)
    return out.reshape(n, 32)


# speedup vs baseline: 1.2528x; 1.2528x over previous
import functools

import numpy as np
import jax
import jax.numpy as jnp
from jax import lax
from jax.experimental import pallas as pl
from jax.experimental.pallas import tpu as pltpu
from jax.experimental.pallas import tpu_sc as plsc

_N_LEVELS = 16
_BASE_RES = 16
_MAX_PARAMS = 2 ** 19
_B_SCALE = 1.3195079565048218
_P1 = int(np.uint32(2654435761).view(np.int32))
_P2 = int(np.uint32(805459861).view(np.int32))
_HASH_MASK = _MAX_PARAMS - 1


def _levels():
    out = []
    off = 0
    for i in range(_N_LEVELS):
        scale = _BASE_RES * np.exp(i * np.log(_B_SCALE)) - 1.0
        res = int(np.ceil(scale)) + 1
        params = res ** 3
        if params % 8 != 0:
            params = ((params + 7) // 8) * 8
        params = min(_MAX_PARAMS, params)
        dense = res ** 3 <= params
        out.append((np.float32(scale), res, params, off, dense))
        off += params
    return out, off


_LEVEL_META, _TOTAL_ROWS = _levels()

_NW = 32
_B = 512
_G = _B // 16

_N_DENSE = sum(1 for m in _LEVEL_META if m[4])
_DENSE_FLOATS = 2 * sum(m[2] for m in _LEVEL_META[:_N_DENSE])
_STAGE_CHUNK = 8 * _B
_STAGE_ITERS = -(-_DENSE_FLOATS // _STAGE_CHUNK)
_DENSE_PAD = _STAGE_ITERS * _STAGE_CHUNK


def _encode_body(px_hbm, py_hbm, pz_hbm, tab_hbm, out_hbm,
                 xs_v, ys_v, zs_v,
                 idx_a, idx_b, w_a, w_b, w_c,
                 rows_a, rows_b,
                 out_v, tab_s, sem_a, sem_b):
    wid = lax.axis_index("s") * 2 + lax.axis_index("c")
    sid = lax.axis_index("s")

    def stage_body(k, _):
        @pl.when(lax.rem(k, 16) == sid)
        def _():
            pltpu.sync_copy(tab_hbm.at[pl.ds(k * _STAGE_CHUNK, _STAGE_CHUNK)],
                            rows_a.at[pl.ds(0, _STAGE_CHUNK)])
            pltpu.sync_copy(rows_a.at[pl.ds(0, _STAGE_CHUNK)],
                            tab_s.at[pl.ds(k * _STAGE_CHUNK, _STAGE_CHUNK)])
        return 0

    lax.fori_loop(0, _STAGE_ITERS, stage_body, 0, unroll=False)
    plsc.subcore_barrier()

    n = px_hbm.shape[0]
    npw = n // _NW
    nchunks = npw // _B
    lane = lax.iota(jnp.int32, 16)
    idx_bufs = (idx_a, idx_b)
    w_bufs = (w_a, w_b, w_c)
    row_bufs = (rows_a, rows_b)
    sems = (sem_a, sem_b)
    half = lax.shift_right_logical(lane, 1)
    par = lane & 1

    def chunk_body(i, _):
        base = wid * npw + i * _B
        pltpu.sync_copy(px_hbm.at[pl.ds(base, _B)], xs_v)
        pltpu.sync_copy(py_hbm.at[pl.ds(base, _B)], ys_v)
        pltpu.sync_copy(pz_hbm.at[pl.ds(base, _B)], zs_v)

        def idx_part(level, g):
            scale, res, size, off, dense = _LEVEL_META[level]
            idx_v = idx_bufs[level % 2]
            w_v = w_bufs[level % 3]
            s = g * 16
            x = xs_v[pl.ds(s, 16)]
            y = ys_v[pl.ds(s, 16)]
            z = zs_v[pl.ds(s, 16)]
            fscale = jnp.float32(scale)
            pxv = x * fscale + jnp.float32(0.5)
            pyv = y * fscale + jnp.float32(0.5)
            pzv = z * fscale + jnp.float32(0.5)
            ix = pxv.astype(jnp.int32)
            iy = pyv.astype(jnp.int32)
            iz = pzv.astype(jnp.int32)
            fx = pxv - ix.astype(jnp.float32)
            fy = pyv - iy.astype(jnp.float32)
            fz = pzv - iz.astype(jnp.float32)
            wx = (jnp.float32(1.0) - fx, fx)
            wy = (jnp.float32(1.0) - fy, fy)
            wz = (jnp.float32(1.0) - fz, fz)
            if dense:
                res2 = res * res
                cx = (ix + off, ix + off + 1)
                ty = (iy * res, iy * res + res)
                tz = (iz * res2, iz * res2 + res2)
                lim = size + off
            else:
                cx = (ix, ix + 1)
                ty = (iy * _P1, iy * _P1 + _P1)
                tz = (iz * _P2, iz * _P2 + _P2)
            lane2 = lane * 2
            for corner in range(8):
                bx, by, bz = corner & 1, (corner >> 1) & 1, (corner >> 2) & 1
                if dense:
                    h = cx[bx] + ty[by] + tz[bz]
                    row = jnp.where(h >= lim, h - size, h)
                else:
                    h = cx[bx] ^ ty[by] ^ tz[bz]
                    row = (h & _HASH_MASK) + off
                w = wx[bx] * wy[by] * wz[bz]
                el0 = row * 2
                if dense:
                    idx_v[pl.ds(corner * _B + s, 16)] = el0
                    idx_v[pl.ds(8 * _B + corner * _B + s, 16)] = el0 + 1
                else:
                    base2 = 2 * (corner * _B + s)
                    plsc.store_scatter(idx_v, [base2 + lane2], el0)
                    plsc.store_scatter(idx_v, [base2 + lane2 + 1], el0 + 1)
                w_v[pl.ds(corner * _B + s, 16)] = w

        def red_part(level, g):
            rows_v = row_bufs[level % 2]
            w_v = w_bufs[level % 3]
            dense = _LEVEL_META[level][4]
            s = g * 16
            acc0 = jnp.zeros((16,), jnp.float32)
            acc1 = jnp.zeros((16,), jnp.float32)
            if dense:
                for corner in range(8):
                    f0 = rows_v[pl.ds(corner * _B + s, 16)]
                    f1 = rows_v[pl.ds(8 * _B + corner * _B + s, 16)]
                    w = w_v[pl.ds(corner * _B + s, 16)]
                    acc0 = acc0 + w * f0
                    acc1 = acc1 + w * f1
                oidx = (s + lane) * 32 + (2 * level)
                plsc.store_scatter(out_v, [oidx], acc0)
                plsc.store_scatter(out_v, [oidx + 1], acc1)
            else:
                for corner in range(8):
                    q = 2 * (corner * _B + s)
                    v0 = rows_v[pl.ds(q, 16)]
                    v1 = rows_v[pl.ds(q + 16, 16)]
                    pb = corner * _B + s
                    wp0 = plsc.load_gather(w_v, [pb + half])
                    wp1 = plsc.load_gather(w_v, [pb + 8 + half])
                    acc0 = acc0 + wp0 * v0
                    acc1 = acc1 + wp1 * v1
                oidx = (s + half) * 32 + (2 * level) + par
                plsc.store_scatter(out_v, [oidx], acc0)
                plsc.store_scatter(out_v, [oidx + 256], acc1)

        def issue(level):
            b = level % 2
            dense = _LEVEL_META[level][4]
            idx_v = idx_bufs[b]
            rows_v = row_bufs[b]
            if dense:
                cp0 = pltpu.async_copy(
                    tab_s.at[idx_v.at[pl.ds(0, 8 * _B)]],
                    rows_v.at[pl.ds(0, 8 * _B)], sems[b])
                cp1 = pltpu.async_copy(
                    tab_s.at[idx_v.at[pl.ds(8 * _B, 8 * _B)]],
                    rows_v.at[pl.ds(8 * _B, 8 * _B)], sems[b])
                return (cp0, cp1)
            cp0 = pltpu.async_copy(tab_hbm.at[idx_v], rows_v, sems[b])
            return (cp0,)

        def run_loop(red_level, idx_level):
            def _b(g):
                if red_level is not None:
                    red_part(red_level, g)
                if idx_level is not None:
                    idx_part(idx_level, g)

            plsc.parallel_loop(0, _G, 1, unroll=1)(_b)

        run_loop(None, 0)
        cp = issue(0)
        run_loop(None, 1)
        cp_next = issue(1)
        for level in range(_N_LEVELS):
            for c in cp:
                c.wait()
            run_loop(level, level + 2 if level + 2 < _N_LEVELS else None)
            if level + 2 < _N_LEVELS:
                cp = cp_next
                cp_next = issue(level + 2)
            elif level + 1 < _N_LEVELS:
                cp = cp_next

        pltpu.sync_copy(out_v, out_hbm.at[pl.ds(base * 32, _B * 32)])
        return 0

    lax.fori_loop(0, nchunks, chunk_body, 0, unroll=False)


def kernel(positions, hash_table):
    n = positions.shape[0]
    px = positions[:, 0]
    py = positions[:, 1]
    pz = positions[:, 2]

    mesh = plsc.VectorSubcoreMesh(core_axis_name="c", subcore_axis_name="s")
    run = functools.partial(
        pl.kernel,
        mesh=mesh,
        compiler_params=pltpu.CompilerParams(needs_layout_passes=False),
        out_type=jax.ShapeDtypeStruct((n * 32,), jnp.float32),
        scratch_types=[
            pltpu.VMEM((_B,), jnp.float32),
            pltpu.VMEM((_B,), jnp.float32),
            pltpu.VMEM((_B,), jnp.float32),
            pltpu.VMEM((16 * _B,), jnp.int32),
            pltpu.VMEM((16 * _B,), jnp.int32),
            pltpu.VMEM((8 * _B,), jnp.float32),
            pltpu.VMEM((8 * _B,), jnp.float32),
            pltpu.VMEM((8 * _B,), jnp.float32),
            pltpu.VMEM((16 * _B,), jnp.float32),
            pltpu.VMEM((16 * _B,), jnp.float32),
            pltpu.VMEM((_B * 32,), jnp.float32),
            pltpu.VMEM_SHARED((_DENSE_PAD,), jnp.float32),
            pltpu.SemaphoreType.DMA,
            pltpu.SemaphoreType.DMA,
        ],
    )(_encode_body)
    out = run(px, py, pz, hash_table)
    return out.reshape(n, 32)


# speedup vs baseline: 1.2539x; 1.0009x over previous
import functools

import numpy as np
import jax
import jax.numpy as jnp
from jax import lax
from jax.experimental import pallas as pl
from jax.experimental.pallas import tpu as pltpu
from jax.experimental.pallas import tpu_sc as plsc

_N_LEVELS = 16
_BASE_RES = 16
_MAX_PARAMS = 2 ** 19
_B_SCALE = 1.3195079565048218
_P1 = int(np.uint32(2654435761).view(np.int32))
_P2 = int(np.uint32(805459861).view(np.int32))
_HASH_MASK = _MAX_PARAMS - 1


def _levels():
    out = []
    off = 0
    for i in range(_N_LEVELS):
        scale = _BASE_RES * np.exp(i * np.log(_B_SCALE)) - 1.0
        res = int(np.ceil(scale)) + 1
        params = res ** 3
        if params % 8 != 0:
            params = ((params + 7) // 8) * 8
        params = min(_MAX_PARAMS, params)
        dense = res ** 3 <= params
        out.append((np.float32(scale), res, params, off, dense))
        off += params
    return out, off


_LEVEL_TABLE, _TOTAL_ROWS = _levels()

_NW = 32
_B = 512
_G = _B // 16

_N_DENSE = sum(1 for m in _LEVEL_TABLE if m[4])
_DENSE_FLOATS = 2 * sum(m[2] for m in _LEVEL_TABLE[:_N_DENSE])
_STAGE_CHUNK = 8 * _B
_STAGE_ITERS = -(-_DENSE_FLOATS // _STAGE_CHUNK)
_DENSE_PAD = _STAGE_ITERS * _STAGE_CHUNK


def _encode_body(px_hbm, py_hbm, pz_hbm, tab_hbm, out_hbm,
                 xs_v, ys_v, zs_v,
                 idx_a, idx_b, w_a, w_b, w_c,
                 rows_a, rows_b,
                 out_v, tab_s, sem_a, sem_b):
    wid = lax.axis_index("s") * 2 + lax.axis_index("c")
    sid = lax.axis_index("s")

    def stage_body(k, _):
        @pl.when(lax.rem(k, 16) == sid)
        def _():
            pltpu.sync_copy(tab_hbm.at[pl.ds(k * _STAGE_CHUNK, _STAGE_CHUNK)],
                            rows_a.at[pl.ds(0, _STAGE_CHUNK)])
            pltpu.sync_copy(rows_a.at[pl.ds(0, _STAGE_CHUNK)],
                            tab_s.at[pl.ds(k * _STAGE_CHUNK, _STAGE_CHUNK)])
        return 0

    lax.fori_loop(0, _STAGE_ITERS, stage_body, 0, unroll=False)
    plsc.subcore_barrier()

    n = px_hbm.shape[0]
    npw = n // _NW
    nchunks = npw // _B
    lane = lax.iota(jnp.int32, 16)
    idx_bufs = (idx_a, idx_b)
    w_bufs = (w_a, w_b, w_c)
    row_bufs = (rows_a, rows_b)
    sems = (sem_a, sem_b)
    half = lax.shift_right_logical(lane, 1)
    par = lane & 1

    def chunk_body(i, _):
        base = wid * npw + i * _B
        pltpu.sync_copy(px_hbm.at[pl.ds(base, _B)], xs_v)
        pltpu.sync_copy(py_hbm.at[pl.ds(base, _B)], ys_v)
        pltpu.sync_copy(pz_hbm.at[pl.ds(base, _B)], zs_v)

        def idx_part(level, g):
            scale, res, size, off, dense = _LEVEL_TABLE[level]
            idx_v = idx_bufs[level % 2]
            w_v = w_bufs[level % 3]
            s = g * 16
            x = xs_v[pl.ds(s, 16)]
            y = ys_v[pl.ds(s, 16)]
            z = zs_v[pl.ds(s, 16)]
            fscale = jnp.float32(scale)
            pxv = x * fscale + jnp.float32(0.5)
            pyv = y * fscale + jnp.float32(0.5)
            pzv = z * fscale + jnp.float32(0.5)
            ix = pxv.astype(jnp.int32)
            iy = pyv.astype(jnp.int32)
            iz = pzv.astype(jnp.int32)
            fx = pxv - ix.astype(jnp.float32)
            fy = pyv - iy.astype(jnp.float32)
            fz = pzv - iz.astype(jnp.float32)
            wx = (jnp.float32(1.0) - fx, fx)
            wy = (jnp.float32(1.0) - fy, fy)
            wz = (jnp.float32(1.0) - fz, fz)
            if dense:
                res2 = res * res
                cx = (ix + off, ix + off + 1)
                ty = (iy * res, iy * res + res)
                tz = (iz * res2, iz * res2 + res2)
                lim = size + off
            else:
                cx = (ix, ix + 1)
                ty = (iy * _P1, iy * _P1 + _P1)
                tz = (iz * _P2, iz * _P2 + _P2)
            lane2 = lane * 2
            for corner in range(8):
                bx, by, bz = corner & 1, (corner >> 1) & 1, (corner >> 2) & 1
                if dense:
                    h = cx[bx] + ty[by] + tz[bz]
                    row = jnp.where(h >= lim, h - size, h)
                else:
                    h = cx[bx] ^ ty[by] ^ tz[bz]
                    row = (h & _HASH_MASK) + off
                w = wx[bx] * wy[by] * wz[bz]
                el0 = row * 2
                if dense:
                    idx_v[pl.ds(corner * _B + s, 16)] = el0
                    idx_v[pl.ds(8 * _B + corner * _B + s, 16)] = el0 + 1
                else:
                    base2 = 2 * (corner * _B + s)
                    plsc.store_scatter(idx_v, [base2 + lane2], el0)
                    plsc.store_scatter(idx_v, [base2 + lane2 + 1], el0 + 1)
                w_v[pl.ds(corner * _B + s, 16)] = w

        def red_part(level, g):
            rows_v = row_bufs[level % 2]
            w_v = w_bufs[level % 3]
            dense = _LEVEL_TABLE[level][4]
            s = g * 16
            acc0 = jnp.zeros((16,), jnp.float32)
            acc1 = jnp.zeros((16,), jnp.float32)
            if dense:
                for corner in range(8):
                    f0 = rows_v[pl.ds(corner * _B + s, 16)]
                    f1 = rows_v[pl.ds(8 * _B + corner * _B + s, 16)]
                    w = w_v[pl.ds(corner * _B + s, 16)]
                    acc0 = acc0 + w * f0
                    acc1 = acc1 + w * f1
                oidx = (s + lane) * 32 + (2 * level)
                plsc.store_scatter(out_v, [oidx], acc0)
                plsc.store_scatter(out_v, [oidx + 1], acc1)
            else:
                for corner in range(8):
                    q = 2 * (corner * _B + s)
                    v0 = rows_v[pl.ds(q, 16)]
                    v1 = rows_v[pl.ds(q + 16, 16)]
                    pb = corner * _B + s
                    wp0 = plsc.load_gather(w_v, [pb + half])
                    wp1 = plsc.load_gather(w_v, [pb + 8 + half])
                    acc0 = acc0 + wp0 * v0
                    acc1 = acc1 + wp1 * v1
                oidx = (s + half) * 32 + (2 * level) + par
                plsc.store_scatter(out_v, [oidx], acc0)
                plsc.store_scatter(out_v, [oidx + 256], acc1)

        def issue(level):
            b = level % 2
            dense = _LEVEL_TABLE[level][4]
            idx_v = idx_bufs[b]
            rows_v = row_bufs[b]
            if dense:
                cp0 = pltpu.async_copy(
                    tab_s.at[idx_v.at[pl.ds(0, 8 * _B)]],
                    rows_v.at[pl.ds(0, 8 * _B)], sems[b])
                cp1 = pltpu.async_copy(
                    tab_s.at[idx_v.at[pl.ds(8 * _B, 8 * _B)]],
                    rows_v.at[pl.ds(8 * _B, 8 * _B)], sems[b])
                return (cp0, cp1)
            cp0 = pltpu.async_copy(tab_hbm.at[idx_v], rows_v, sems[b])
            return (cp0,)

        def run_loop(red_level, idx_level):
            def _b(g):
                if red_level is not None:
                    red_part(red_level, g)
                if idx_level is not None:
                    idx_part(idx_level, g)

            plsc.parallel_loop(0, _G, 1, unroll=1)(_b)

        run_loop(None, 0)
        cp = issue(0)
        run_loop(None, 1)
        cp_next = issue(1)
        for level in range(_N_LEVELS):
            for c in cp:
                c.wait()
            run_loop(level, level + 2 if level + 2 < _N_LEVELS else None)
            if level + 2 < _N_LEVELS:
                cp = cp_next
                cp_next = issue(level + 2)
            elif level + 1 < _N_LEVELS:
                cp = cp_next

        pltpu.sync_copy(out_v, out_hbm.at[pl.ds(base * 32, _B * 32)])
        return 0

    lax.fori_loop(0, nchunks, chunk_body, 0, unroll=False)


def kernel(positions, hash_table):
    n = positions.shape[0]
    px = positions[:, 0]
    py = positions[:, 1]
    pz = positions[:, 2]

    mesh = plsc.VectorSubcoreMesh(core_axis_name="c", subcore_axis_name="s")
    run = functools.partial(
        pl.kernel,
        mesh=mesh,
        compiler_params=pltpu.CompilerParams(needs_layout_passes=False),
        out_type=jax.ShapeDtypeStruct((n * 32,), jnp.float32),
        scratch_types=[
            pltpu.VMEM((_B,), jnp.float32),
            pltpu.VMEM((_B,), jnp.float32),
            pltpu.VMEM((_B,), jnp.float32),
            pltpu.VMEM((16 * _B,), jnp.int32),
            pltpu.VMEM((16 * _B,), jnp.int32),
            pltpu.VMEM((8 * _B,), jnp.float32),
            pltpu.VMEM((8 * _B,), jnp.float32),
            pltpu.VMEM((8 * _B,), jnp.float32),
            pltpu.VMEM((16 * _B,), jnp.float32),
            pltpu.VMEM((16 * _B,), jnp.float32),
            pltpu.VMEM((_B * 32,), jnp.float32),
            pltpu.VMEM_SHARED((_DENSE_PAD,), jnp.float32),
            pltpu.SemaphoreType.DMA,
            pltpu.SemaphoreType.DMA,
        ],
    )(_encode_body)
    out = run(px, py, pz, hash_table)
    return out.reshape(n, 32)
